# baseline (speedup 1.0000x reference)
import functools

import jax
import jax.numpy as jnp
from jax import lax
from jax.experimental import pallas as pl
from jax.experimental.pallas import tpu as pltpu
from jax.experimental.pallas import tpu_sc as plsc

VOCAB = 1000000
DIM = 64
BATCH = 4096
HIST = 200

NC = 2
NS = 16
NW = NC * NS
DOCS_PER_W = BATCH // NW
HALF = HIST // 2
GW = 128
N_CHUNKS = DOCS_PER_W
OUT_RING = 16
UNROLL = 4


@functools.cache
def _build():
  mesh = plsc.VectorSubcoreMesh(core_axis_name="c", subcore_axis_name="s")

  @functools.partial(
      pl.kernel,
      mesh=mesh,
      compiler_params=pltpu.CompilerParams(use_tc_tiling_on_sc=True),
      out_type=jax.ShapeDtypeStruct((BATCH, 4 * DIM), jnp.float32),
      scratch_types=[
          pltpu.VMEM((DOCS_PER_W * 2 * GW,), jnp.int32),
          pltpu.VMEM((2, 2 * GW, 2 * DIM), jnp.float32),
          pltpu.VMEM((OUT_RING, 4 * DIM), jnp.float32),
          pltpu.SemaphoreType.DMA,
          pltpu.SemaphoreType.DMA,
      ],
  )
  def _sc_embed_pool(table_h, idx_h, out_h, idx_v, rows_v, out_v, sem0, sem1):
    wid = lax.axis_index("s") * NC + lax.axis_index("c")

    nw_idx = DOCS_PER_W * 2 * GW
    pltpu.sync_copy(idx_h.at[pl.ds(wid * nw_idx, nw_idx)], idx_v)

    def issue(c, b):
        sem = sem0 if b == 0 else sem1
        for h in range(2):
            pltpu.async_copy(
                table_h.at[idx_v.at[pl.ds((c * 2 + h) * GW, GW)]],
                rows_v.at[b, pl.ds(h * GW, GW)],
                sem,
            )

    def drain(c, b):
        sem = sem0 if b == 0 else sem1
        for h in range(2):
            pltpu.make_async_copy(
                table_h.at[idx_v.at[pl.ds((c * 2 + h) * GW, GW)]],
                rows_v.at[b, pl.ds(h * GW, GW)],
                sem,
            ).wait()

    def reduce_doc(c, b):
        rb = rows_v.at[b]

        def make_body(base):
            def body(i, carry):
                mx, mn, sm = carry
                for u in range(UNROLL):
                    r = base + i * UNROLL + u
                    for k in range(4):
                        v = rb[r, pl.ds(k * 16, 16)]
                        mx = tuple(jnp.maximum(mx[q], v) if q == k else mx[q]
                                   for q in range(4))
                        mn = tuple(jnp.minimum(mn[q], v) if q == k else mn[q]
                                   for q in range(4))
                        sm = tuple(sm[q] + v if q == k else sm[q]
                                   for q in range(4))
                return (mx, mn, sm)
            return body

        neg = jnp.full((16,), -jnp.inf, jnp.float32)
        pos = jnp.full((16,), jnp.inf, jnp.float32)
        zero = jnp.zeros((16,), jnp.float32)
        carry = ((neg,) * 4, (pos,) * 4, (zero,) * 4)
        carry = lax.fori_loop(0, HALF // UNROLL, make_body(0), carry)
        mx, mn, sm = lax.fori_loop(0, HALF // UNROLL, make_body(GW), carry)

        slot = c % OUT_RING
        inv = jnp.float32(1.0 / HIST)
        for k in range(4):
            out_v[slot, pl.ds(k * 16, 16)] = mx[k]
            out_v[slot, pl.ds(DIM + k * 16, 16)] = mn[k]
            out_v[slot, pl.ds(2 * DIM + k * 16, 16)] = sm[k] * inv
            out_v[slot, pl.ds(3 * DIM + k * 16, 16)] = sm[k]

    issue(0, 0)
    issue(1, 1)

    def outer(g, carry):
        for b in range(2):
            c = 2 * g + b
            drain(c, b)
            reduce_doc(c, b)

            @pl.when(c < N_CHUNKS - 2)
            def _():
                issue(c + 2, b)

            @pl.when(c % OUT_RING == OUT_RING - 1)
            def _():
                row0 = pl.multiple_of(
                    wid * DOCS_PER_W + c - (OUT_RING - 1), 8)
                pltpu.sync_copy(out_v, out_h.at[pl.ds(row0, OUT_RING)])
        return carry

    lax.fori_loop(0, N_CHUNKS // 2, outer, 0)

  return _sc_embed_pool


def kernel(table, indices):
    t2 = jnp.pad(table, ((0, 0), (0, DIM)))
    idx = indices.astype(jnp.int32).reshape(BATCH * 2, HALF)
    idxp = jnp.concatenate([idx, idx[:, :GW - HALF]], axis=1).reshape(-1)
    return _build()(t2, idxp)


# speedup vs baseline: 1.1081x; 1.1081x over previous
import functools

import jax
import jax.numpy as jnp
from jax import lax
from jax.experimental import pallas as pl
from jax.experimental.pallas import tpu as pltpu
from jax.experimental.pallas import tpu_sc as plsc

VOCAB = 1000000
DIM = 64
BATCH = 4096
HIST = 200

NC = 2
NS = 16
NW = NC * NS
DOCS_PER_W = BATCH // NW
HALF = HIST // 2
CHUNK_DOCS = 2
N_CHUNKS = DOCS_PER_W // CHUNK_DOCS
ROWS_PER_CHUNK = CHUNK_DOCS * HIST
GATHERS_PER_CHUNK = CHUNK_DOCS * 2
UNROLL = 4

@functools.cache
def _build():
  mesh = plsc.VectorSubcoreMesh(core_axis_name="c", subcore_axis_name="s")

  @functools.partial(
      pl.kernel,
      mesh=mesh,
      compiler_params=pltpu.CompilerParams(use_tc_tiling_on_sc=False),
      out_type=jax.ShapeDtypeStruct((BATCH, 4 * DIM), jnp.float32),
      scratch_types=[
          pltpu.VMEM((DOCS_PER_W * 2, HALF), jnp.int32),
          pltpu.VMEM((2, ROWS_PER_CHUNK, DIM), jnp.float32),
          pltpu.VMEM((DOCS_PER_W, 4 * DIM), jnp.float32),
          pltpu.SemaphoreType.DMA,
          pltpu.SemaphoreType.DMA,
      ],
  )
  def _sc_embed_pool(table_h, idx_h, out_h, idx_v, rows_v, out_v, sem0, sem1):
    wid = lax.axis_index("s") * NC + lax.axis_index("c")

    pltpu.sync_copy(idx_h.at[pl.ds(wid * (DOCS_PER_W * 2), DOCS_PER_W * 2)],
                    idx_v)

    def issue(c, b):
        sem = sem0 if b == 0 else sem1
        for k in range(GATHERS_PER_CHUNK):
            pltpu.async_copy(
                table_h.at[idx_v.at[c * GATHERS_PER_CHUNK + k]],
                rows_v.at[b, pl.ds(k * HALF, HALF)],
                sem,
            )

    def drain(c, b):
        sem = sem0 if b == 0 else sem1
        for k in range(GATHERS_PER_CHUNK):
            pltpu.make_async_copy(
                table_h.at[idx_v.at[c * GATHERS_PER_CHUNK + k]],
                rows_v.at[b, pl.ds(k * HALF, HALF)],
                sem,
            ).wait()

    def reduce_doc(c, b, j):
        rb = rows_v.at[b]
        base = j * HIST

        def body(i, carry):
            mx, mn, sm = carry
            for u in range(UNROLL):
                r = base + i * UNROLL + u
                for k in range(4):
                    v = rb[r, pl.ds(k * 16, 16)]
                    mx = tuple(jnp.maximum(mx[q], v) if q == k else mx[q]
                               for q in range(4))
                    mn = tuple(jnp.minimum(mn[q], v) if q == k else mn[q]
                               for q in range(4))
                    sm = tuple(sm[q] + v if q == k else sm[q]
                               for q in range(4))
            return (mx, mn, sm)

        neg = jnp.full((16,), -jnp.inf, jnp.float32)
        pos = jnp.full((16,), jnp.inf, jnp.float32)
        zero = jnp.zeros((16,), jnp.float32)
        mx, mn, sm = lax.fori_loop(
            0, HIST // UNROLL, body,
            ((neg,) * 4, (pos,) * 4, (zero,) * 4))

        doc = c * CHUNK_DOCS + j
        inv = jnp.float32(1.0 / HIST)
        for k in range(4):
            out_v[doc, pl.ds(k * 16, 16)] = mx[k]
            out_v[doc, pl.ds(DIM + k * 16, 16)] = mn[k]
            out_v[doc, pl.ds(2 * DIM + k * 16, 16)] = sm[k] * inv
            out_v[doc, pl.ds(3 * DIM + k * 16, 16)] = sm[k]

    issue(0, 0)
    issue(1, 1)

    def outer(g, carry):
        for b in range(2):
            c = 2 * g + b
            drain(c, b)
            reduce_doc(c, b, 0)
            reduce_doc(c, b, 1)

            @pl.when(g < (N_CHUNKS // 2 - 1))
            def _():
                issue(c + 2, b)
        return carry

    lax.fori_loop(0, N_CHUNKS // 2, outer, 0)

    pltpu.sync_copy(out_v, out_h.at[pl.ds(wid * DOCS_PER_W, DOCS_PER_W)])

  return _sc_embed_pool


def kernel(table, indices):
    idx = indices.astype(jnp.int32).reshape(BATCH * 2, HALF)
    return _build()(table, idx)
